# baseline (speedup 1.0000x reference)
import functools

import jax
import jax.numpy as jnp
from jax import lax
from jax.experimental import pallas as pl
from jax.experimental.pallas import tpu as pltpu
from jax.experimental.pallas import tpu_sc as plsc

_NC = 2
_NS = 16
_NW = _NC * _NS
_LANES = 16
_NBUF = 2


def kernel(x, token_table, pos_table):
    B, S = x.shape
    V, E = token_table.shape
    RPW = B // _NW

    SBLK = -(-S // _LANES)
    SPAD = SBLK * _LANES
    SPK = SPAD // 2
    NCH = -(-S // 128)
    SIDX = NCH * 128

    tok2 = token_table.reshape(V // 2, 2 * E)
    x2 = (x >> 1).astype(jnp.int32)
    x2 = jnp.pad(x2, ((0, 0), (0, SIDX - S))).reshape(B * SIDX)
    xp = jnp.pad((x & 1).astype(jnp.int32), ((0, 0), (0, SPAD - S)))
    pmask = jnp.sum(xp.reshape(B * SBLK, _LANES)
                    << jnp.arange(_LANES, dtype=jnp.int32)[None, :],
                    axis=1).astype(jnp.int32)
    NMR = RPW * SBLK // _LANES
    pmask = pmask.reshape(_NW, NMR, _LANES)
    pos2 = jnp.pad(pos_table[:S], ((0, SPAD - S), (0, 0))).reshape(SPK, 2 * E)

    mesh = plsc.VectorSubcoreMesh(core_axis_name="c", subcore_axis_name="s")

    @functools.partial(
        pl.kernel,
        out_type=jax.ShapeDtypeStruct((B, SPK, 2 * E), jnp.float32),
        mesh=mesh,
        scratch_types=[
            pltpu.VMEM((RPW * SIDX,), jnp.int32),
            pltpu.VMEM((NMR, _LANES), jnp.int32),
            pltpu.SMEM((RPW * SBLK,), jnp.int32),
            pltpu.VMEM((SPK, 2 * E), jnp.float32),
            pltpu.VMEM((_NBUF, SIDX, 2 * E), jnp.float32),
            pltpu.VMEM((_NBUF, SPK, 2 * E), jnp.float32),
            pltpu.SemaphoreType.DMA((_NBUF,)),
            pltpu.SemaphoreType.DMA((_NBUF,)),
        ],
        compiler_params=pltpu.CompilerParams(use_tc_tiling_on_sc=True),
    )
    def emb(x2_hbm, pm_hbm, tok_hbm, pos_hbm, out_hbm,
            idx_v, pm_v, pm_s, pos_v, bufs, obufs, gsem, osem):
        wid = lax.axis_index("c") * _NS + lax.axis_index("s")
        base = wid * RPW
        pltpu.sync_copy(x2_hbm.at[pl.ds(base * SIDX, RPW * SIDX)], idx_v)
        pltpu.sync_copy(pm_hbm.at[wid], pm_v)
        pltpu.sync_copy(pos_hbm, pos_v)

        @pl.loop(0, NMR)
        def _spill(sr):
            vec = pm_v[sr]
            for t in range(_LANES):
                pm_s[sr * _LANES + t] = vec[t]

        def fire_gather(m, b):
            for h in range(NCH):
                pltpu.async_copy(
                    tok_hbm.at[idx_v.at[pl.ds(m * SIDX + h * 128, 128)]],
                    bufs.at[b, pl.ds(h * 128, 128)], gsem.at[b])

        def wait_gather(b):
            for h in range(NCH):
                pltpu.make_async_copy(
                    tok_hbm.at[pl.ds(0, 128)],
                    bufs.at[b, pl.ds(h * 128, 128)], gsem.at[b]).wait()

        def fire_out(j, b):
            pltpu.async_copy(obufs.at[b], out_hbm.at[base + j], osem.at[b])

        def wait_out(b):
            pltpu.make_async_copy(
                obufs.at[b], out_hbm.at[base], osem.at[b]).wait()

        for b in range(_NBUF):
            fire_gather(b, b)

        @pl.loop(0, RPW // _NBUF)
        def _ring(g):
            for b in range(_NBUF):
                j = g * _NBUF + b
                bp = (b - 1) % _NBUF
                m = j + _NBUF - 1

                @pl.when(jnp.logical_and(j >= 1, m < RPW))
                def _():
                    wait_out(bp)
                    fire_gather(m, bp)

                wait_gather(b)

                @pl.loop(0, SBLK)
                def _add(blk):
                    r0 = blk * _LANES
                    pm = pm_s[j * SBLK + blk]
                    for i in range(_LANES):
                        r = r0 + i
                        hi = ((pm >> i) & 1) != 0
                        pr = blk * (_LANES // 2) + i // 2
                        co = (i % 2) * E
                        for c in range(0, E, _LANES):
                            va = bufs[b, r, pl.ds(c, _LANES)]
                            vb = bufs[b, r, pl.ds(E + c, _LANES)]
                            obufs[b, pr, pl.ds(co + c, _LANES)] = (
                                jnp.where(hi, vb, va)
                                + pos_v[pr, pl.ds(co + c, _LANES)])

                fire_out(j, b)

        for b in range(_NBUF):
            wait_out(b)

    out = emb(x2, pmask, tok2, pos2)
    return out.reshape(B, SPAD, E)[:, :S].reshape(B, S, E)
